# baseline (speedup 1.0000x reference)
import jax
import jax.numpy as jnp
from jax.experimental import pallas as pl
from jax.experimental.pallas import tpu as pltpu

_E = 320000


def _merge_body(gs, gt, gw, hs, ht, emb, out_s, out_t, out_w, out_e,
                s_v, t_v, w_v, hs_v, ht_v, emb_v, sem_in, sem_out):
    top = pl.ds(0, _E)
    bot = pl.ds(_E, _E)
    loads = [
        pltpu.make_async_copy(hs, hs_v, sem_in.at[0]),
        pltpu.make_async_copy(ht, ht_v, sem_in.at[1]),
        pltpu.make_async_copy(gw, w_v.at[top], sem_in.at[2]),
        pltpu.make_async_copy(gs, s_v.at[top], sem_in.at[3]),
        pltpu.make_async_copy(gt, t_v.at[top], sem_in.at[4]),
        pltpu.make_async_copy(emb, emb_v, sem_in.at[5]),
    ]
    for h in loads:
        h.start()

    w_v[bot] = jnp.ones((_E,), jnp.float32)
    loads[2].wait()
    store_w = pltpu.make_async_copy(w_v, out_w, sem_out.at[0])
    store_w.start()

    loads[0].wait()
    s_v[bot] = hs_v[...]
    loads[3].wait()
    store_s = pltpu.make_async_copy(s_v, out_s, sem_out.at[1])
    store_s.start()

    loads[1].wait()
    t_v[bot] = ht_v[...]
    loads[4].wait()
    store_t = pltpu.make_async_copy(t_v, out_t, sem_out.at[2])
    store_t.start()

    loads[5].wait()
    store_e = pltpu.make_async_copy(emb_v, out_e, sem_out.at[3])
    store_e.start()

    store_w.wait()
    store_s.wait()
    store_t.wait()
    store_e.wait()


def kernel(gen_sources, gen_targets, gen_weights, given_sources, given_targets, node_embeddings):
    hbm = pl.BlockSpec(memory_space=pltpu.MemorySpace.HBM)
    out_s, out_t, out_w, out_e = pl.pallas_call(
        _merge_body,
        in_specs=[hbm] * 6,
        out_specs=[hbm] * 4,
        out_shape=(
            jax.ShapeDtypeStruct((2 * _E,), jnp.int32),
            jax.ShapeDtypeStruct((2 * _E,), jnp.int32),
            jax.ShapeDtypeStruct((2 * _E,), jnp.float32),
            jax.ShapeDtypeStruct(node_embeddings.shape, node_embeddings.dtype),
        ),
        scratch_shapes=[
            pltpu.VMEM((2 * _E,), jnp.int32),
            pltpu.VMEM((2 * _E,), jnp.int32),
            pltpu.VMEM((2 * _E,), jnp.float32),
            pltpu.VMEM((_E,), jnp.int32),
            pltpu.VMEM((_E,), jnp.int32),
            pltpu.VMEM(node_embeddings.shape, node_embeddings.dtype),
            pltpu.SemaphoreType.DMA((6,)),
            pltpu.SemaphoreType.DMA((4,)),
        ],
    )(gen_sources, gen_targets, gen_weights, given_sources, given_targets, node_embeddings)
    return out_s, out_t, out_w, out_e
